# Initial kernel scaffold; baseline (speedup 1.0000x reference)
import jax
import jax.numpy as jnp
from jax.experimental import pallas as pl


def kernel(classifications, bboxes):
    raise NotImplementedError("write your pallas kernel here")


# speedup vs baseline: 28.3456x; 28.3456x over previous
import jax
import jax.numpy as jnp
from jax.experimental import pallas as pl
from jax.experimental.pallas import tpu as pltpu

_BBOX_NUM = 300
_NMS_THRESHOLD = 0.5
_N = 20000
_N_PAD = 20480
_NEG = -jnp.inf


def _nms_kernel(score_ref, y1_ref, x1_ref, y2_ref, x2_ref, out_ref, live_ref):
    B = score_ref.shape[0]
    live_ref[...] = score_ref[...]
    y1 = y1_ref[...]
    x1 = x1_ref[...]
    y2 = y2_ref[...]
    x2 = x2_ref[...]
    areas = jnp.maximum(y2 - y1, 0.0) * jnp.maximum(x2 - x1, 0.0)
    pos = jax.lax.broadcasted_iota(jnp.int32, (B, _N_PAD), 1)

    def body(k, _):
        live = live_ref[...]
        m = jnp.max(live, axis=1, keepdims=True)
        valid = m > _NEG
        idx = jnp.min(jnp.where(live == m, pos, _N_PAD), axis=1, keepdims=True)
        is_sel = pos == idx
        sel_f = is_sel.astype(jnp.float32)
        by1 = jnp.sum(sel_f * y1, axis=1, keepdims=True)
        bx1 = jnp.sum(sel_f * x1, axis=1, keepdims=True)
        by2 = jnp.sum(sel_f * y2, axis=1, keepdims=True)
        bx2 = jnp.sum(sel_f * x2, axis=1, keepdims=True)
        yy1 = jnp.maximum(by1, y1)
        xx1 = jnp.maximum(bx1, x1)
        yy2 = jnp.minimum(by2, y2)
        xx2 = jnp.minimum(bx2, x2)
        inter = jnp.maximum(yy2 - yy1, 0.0) * jnp.maximum(xx2 - xx1, 0.0)
        area_box = jnp.maximum(by2 - by1, 0.0) * jnp.maximum(bx2 - bx1, 0.0)
        union = area_box + areas - inter
        iou = jnp.where(union > 0.0, inter / union, 0.0)
        live_ref[...] = jnp.where((iou > _NMS_THRESHOLD) | is_sel, _NEG, live)
        row = jnp.concatenate([by1, bx1, by2, bx2], axis=1)
        out_ref[pl.ds(k, 1)] = jnp.where(valid, row, -1.0)[None]
        return 0

    jax.lax.fori_loop(0, _BBOX_NUM, body, 0)


def kernel(classifications, bboxes):
    B = classifications.shape[0]
    scores = classifications[:, :, 1]
    scores = jnp.pad(scores, ((0, 0), (0, _N_PAD - _N)), constant_values=_NEG)
    coords = jnp.pad(bboxes, ((0, 0), (0, _N_PAD - _N), (0, 0)))
    y1 = coords[:, :, 0]
    x1 = coords[:, :, 1]
    y2 = coords[:, :, 2]
    x2 = coords[:, :, 3]
    out = pl.pallas_call(
        _nms_kernel,
        out_shape=jax.ShapeDtypeStruct((_BBOX_NUM, B, 4), jnp.float32),
        scratch_shapes=[pltpu.VMEM((B, _N_PAD), jnp.float32)],
    )(scores, y1, x1, y2, x2)
    return jnp.transpose(out, (1, 0, 2))
